# baseline (speedup 1.0000x reference)
import functools

import jax
import jax.numpy as jnp
from jax import lax
from jax.experimental import pallas as pl
from jax.experimental.pallas import tpu as pltpu
from jax.experimental.pallas import tpu_sc as plsc

H = 128
L = 16
NC = 2
NS = 16
NW = NC * NS
C = 64
S = 8

_PERM_DN = jax.lax.GatherDimensionNumbers(
    offset_dims=(), collapsed_slice_dims=(0,), start_index_map=(0,))


def _perm(v, idx):
    return jax.lax.gather(
        v, idx[:, None], _PERM_DN, (1,),
        mode=jax.lax.GatherScatterMode.PROMISE_IN_BOUNDS,
        unique_indices=True, indices_are_sorted=False)


def _inter(v):
    n = v.shape[0]
    return v.reshape(n, H // 32, 2, L).transpose(0, 1, 3, 2).reshape(n, H)


def _matmul_body(x_ref, w1_ref, b1_ref, xa_ref, xb_ref):
    xv = x_ref[...]
    w = w1_ref[...]
    dn = (((1,), (0,)), ((), ()))
    hi = jax.lax.Precision.HIGHEST
    xa_ref[...] = (
        jax.lax.dot_general(xv, w[:H, :], dn, precision=hi) + b1_ref[...]
    )
    xb_ref[...] = jax.lax.dot_general(xv, w[H:, :], dn, precision=hi)


def _ln_body(x_ref, acc_ref, g_ref, b_ref, o_ref):
    a = x_ref[...] + acc_ref[0] + acc_ref[1]
    mean = jnp.mean(a, axis=-1, keepdims=True)
    d = a - mean
    var = jnp.mean(d * d, axis=-1, keepdims=True)
    o_ref[...] = d * jax.lax.rsqrt(var + 1e-5) * g_ref[...] + b_ref[...]


def _make_sc_kernel(n_nodes, acc_rows, k0, k1):
    stripe = acc_rows // NS
    assert k0 % (2 * S) == 0 and k1 % (2 * S) == 0

    mesh = plsc.VectorSubcoreMesh(core_axis_name="c", subcore_axis_name="s")

    @functools.partial(
        pl.kernel,
        out_type=jax.ShapeDtypeStruct((NC, acc_rows, H), jnp.float32),
        mesh=mesh,
        compiler_params=pltpu.CompilerParams(needs_layout_passes=False),
        scratch_types=[
            pltpu.VMEM((2, C, H), jnp.int32),
            pltpu.VMEM((2, C, H), jnp.float32),
            pltpu.VMEM((2, S, C), jnp.int32),
            pltpu.VMEM((2, S, C), jnp.int32),
            pltpu.VMEM((H // 2, ), jnp.int32),
            pltpu.VMEM((L, ), jnp.float32),
            pltpu.VMEM((8, H), jnp.float32),
            pltpu.VMEM_SHARED((acc_rows, H), jnp.float32),
            pltpu.SemaphoreType.DMA,
            pltpu.SemaphoreType.DMA,
            pltpu.SemaphoreType.DMA,
            pltpu.SemaphoreType.DMA,
            pltpu.SemaphoreType.DMA,
            pltpu.SemaphoreType.DMA,
            pltpu.SemaphoreType.DMA,
            pltpu.SemaphoreType.DMA,
        ],
    )
    def sc_edge(xcat_hbm, xb_hbm, src_hbm, dst_hbm, w2_hbm, pv_hbm, acc_out,
                gbuf, bbuf, sslab, dslab, w2v, pv, zbuf, acc,
                sem_pre, sem_slab, sg0, sg1, sb0, sb1, sem_sc, sem_zero):
        cid = lax.axis_index("c")
        sid = lax.axis_index("s")
        k_mine = jnp.where(cid == 0, k0, k1)
        row0 = jnp.where(cid == 0, sid * k0, NS * k0 + sid * k1)
        sem_g = (sg0, sg1)
        sem_b = (sb0, sb1)

        def _issue_slab(m):
            pltpu.async_copy(src_hbm.at[pl.ds(row0 + m * S, S)],
                             sslab.at[lax.rem(m, 2)], sem_slab)
            pltpu.async_copy(dst_hbm.at[pl.ds(row0 + m * S, S)],
                             dslab.at[lax.rem(m, 2)], sem_slab)

        def _wait_slab(m):
            pltpu.make_async_copy(src_hbm.at[pl.ds(row0 + m * S, S)],
                                  sslab.at[lax.rem(m, 2)], sem_slab).wait()
            pltpu.make_async_copy(dst_hbm.at[pl.ds(row0 + m * S, S)],
                                  dslab.at[lax.rem(m, 2)], sem_slab).wait()

        def _sidx(k):
            return sslab.at[lax.rem(k // S, 2), lax.rem(k, S)]

        def _didx(k):
            return dslab.at[lax.rem(k // S, 2), lax.rem(k, S)]

        def _issue_gathers(k, s):
            pltpu.async_copy(xcat_hbm.at[_sidx(k)], gbuf.at[s], sem_g[s])
            pltpu.async_copy(xb_hbm.at[_didx(k)], bbuf.at[s], sem_b[s])

        def _wait_gathers(k, s):
            pltpu.make_async_copy(xcat_hbm.at[_sidx(k)], gbuf.at[s],
                                  sem_g[s]).wait()
            pltpu.make_async_copy(xb_hbm.at[_didx(k)], bbuf.at[s],
                                  sem_b[s]).wait()

        def _wait_scatter(k, par):
            pltpu.make_async_copy(bbuf.at[par], acc.at[_didx(k)],
                                  sem_sc).wait()

        _issue_slab(0)
        pltpu.async_copy(w2_hbm, w2v, sem_pre)
        pltpu.async_copy(pv_hbm, pv, sem_pre)

        zero16 = jnp.zeros((L,), jnp.float32)
        for i in range(8):
            for j8 in range(H // L):
                zbuf[i, pl.ds(j8 * L, L)] = zero16

        _wait_slab(0)
        _issue_gathers(0, 0)

        def _zero_issue(i, _):
            pltpu.async_copy(zbuf, acc.at[pl.ds(sid * stripe + i * 8, 8)],
                             sem_zero)
            return 0

        def _zero_drain(i, _):
            pltpu.make_async_copy(
                zbuf, acc.at[pl.ds(sid * stripe + i * 8, 8)],
                sem_zero).wait()
            return 0

        lax.fori_loop(0, stripe // 8, _zero_issue, 0)
        lax.fori_loop(0, stripe // 8, _zero_drain, 0)

        pltpu.make_async_copy(w2_hbm, w2v, sem_pre).wait()
        pltpu.make_async_copy(pv_hbm, pv, sem_pre).wait()
        plsc.subcore_barrier()

        b2s = pv[pl.ds(0, L)][0]
        lanes = lax.iota(jnp.int32, L)
        w2blk = [plsc.bitcast(w2v[pl.ds(g * L, L)], jnp.bfloat16)
                 for g in range(H // 32)]

        def _pair_body(kk, _):
            for s in range(2):
                k = kk * 2 + s
                @pl.when(k >= 1)
                def _():
                    _wait_scatter(k - 1, 1 - s)

                @pl.when(jnp.logical_and(lax.rem(k, S) == 0,
                                         k + S < k_mine))
                def _():
                    _issue_slab(k // S + 1)

                @pl.when(k + 1 < k_mine)
                def _():
                    @pl.when(lax.rem(k + 1, S) == 0)
                    def _():
                        _wait_slab((k + 1) // S)

                    _issue_gathers(k + 1, 1 - s)

                _wait_gathers(k, s)

                g2 = gbuf.at[s]
                bb = bbuf.at[s]

                def _zgroup(gr, _):
                    e0 = gr * L
                    zv = jnp.zeros((L,), jnp.float32)
                    for i in range(L):
                        e = e0 + i
                        sacc = None
                        for g in range(H // 32):
                            av = plsc.bitcast(g2[e, pl.ds(g * L, L)],
                                              jnp.bfloat16)
                            blo = bb[e, pl.ds(g * 32, L)]
                            bhi = bb[e, pl.ds(g * 32 + L, L)]
                            bv = plsc.pack(
                                blo, bhi, format=plsc.PackFormat.INTERLEAVED)
                            t = jnp.maximum(av + bv, 0) * w2blk[g]
                            sacc = t if sacc is None else sacc + t
                        sa, sb = plsc.unpack(
                            sacc, format=plsc.PackFormat.INTERLEAVED)
                        sv = sa + sb
                        for sh in (8, 4, 2, 1):
                            sv = sv + _perm(sv, lanes ^ sh)
                        zv = zv + jnp.where(lanes == i, sv, 0.0)
                    wgt = 1.0 / (1.0 + jnp.exp(-(zv + b2s)))
                    for i in range(L):
                        e = e0 + i
                        we = wgt[i]
                        for g in range(H // 32):
                            v32 = plsc.bitcast(
                                g2[e, pl.ds(H // 2 + g * L, L)], jnp.bfloat16)
                            va, vb = plsc.unpack(
                                v32, format=plsc.PackFormat.INTERLEAVED)
                            bb[e, pl.ds(g * 32, L)] = va * we
                            bb[e, pl.ds(g * 32 + L, L)] = vb * we
                    return 0

                lax.fori_loop(0, C // L, _zgroup, 0)

                pltpu.async_copy(bb, acc.at[_didx(k)], sem_sc, add=True)
            return 0

        lax.fori_loop(0, k_mine // 2, _pair_body, 0)

        _wait_scatter(k_mine - 1, 1)
        plsc.subcore_barrier()
        pltpu.sync_copy(
            acc.at[pl.ds(sid * stripe, stripe)],
            acc_out.at[cid, pl.ds(sid * stripe, stripe)],
        )

    return sc_edge


def kernel(x, edge_index, W1, b1, W2, b2, gamma, beta):
    n_nodes = x.shape[0]
    n_edges = edge_index.shape[1]

    src = edge_index[0].astype(jnp.int32)
    dst = edge_index[1].astype(jnp.int32)

    k_chunks = -(-n_edges // (NW * C))
    k_chunks += -k_chunks % (2 * S)
    k0 = ((2 * k_chunks * 35) // 100 // (2 * S)) * (2 * S)
    k1 = 2 * k_chunks - k0
    e_pad = NW * k_chunks * C - n_edges
    srcp = jnp.concatenate([src, jnp.zeros((e_pad,), jnp.int32)])
    dstp = jnp.concatenate([dst, jnp.full((e_pad,), n_nodes, jnp.int32)])
    srcp = srcp.reshape(NW * k_chunks, C)
    dstp = dstp.reshape(NW * k_chunks, C)
    del n_edges

    acc_rows = n_nodes + (-n_nodes % (NS * 8)) + (NS * 8 if n_nodes % (NS * 8) == 0 and e_pad else 0)

    br = 400
    grid = n_nodes // br
    xa, xb = pl.pallas_call(
        _matmul_body,
        grid=(grid,),
        in_specs=[
            pl.BlockSpec((br, H), lambda i: (i, 0)),
            pl.BlockSpec((2 * H, H), lambda i: (0, 0)),
            pl.BlockSpec((1, H), lambda i: (0, 0)),
        ],
        out_specs=[
            pl.BlockSpec((br, H), lambda i: (i, 0)),
            pl.BlockSpec((br, H), lambda i: (i, 0)),
        ],
        out_shape=[
            jax.ShapeDtypeStruct((n_nodes, H), jnp.float32),
            jax.ShapeDtypeStruct((n_nodes, H), jnp.float32),
        ],
    )(x, W1, b1.reshape(1, H))

    def _as_i32(v):
        n = v.shape[0]
        return jax.lax.bitcast_convert_type(
            v.astype(jnp.bfloat16).reshape(n, v.shape[1] // 2, 2), jnp.int32)

    xcat = _as_i32(jnp.concatenate([_inter(xa), _inter(x)], axis=1))
    w2p = _as_i32(_inter(W2.reshape(1, H)))[0]
    pvv = jnp.concatenate([b2.reshape(1), jnp.zeros((L - 1,), jnp.float32)])

    sc_edge = _make_sc_kernel(n_nodes, acc_rows, k0, k1)
    acc = sc_edge(xcat, xb, srcp, dstp, w2p, pvv)

    out = pl.pallas_call(
        _ln_body,
        grid=(grid,),
        in_specs=[
            pl.BlockSpec((br, H), lambda i: (i, 0)),
            pl.BlockSpec((NC, br, H), lambda i: (0, i, 0)),
            pl.BlockSpec((1, H), lambda i: (0, 0)),
            pl.BlockSpec((1, H), lambda i: (0, 0)),
        ],
        out_specs=pl.BlockSpec((br, H), lambda i: (i, 0)),
        out_shape=jax.ShapeDtypeStruct((n_nodes, H), jnp.float32),
    )(x, acc, gamma.reshape(1, H), beta.reshape(1, H))

    return out


# speedup vs baseline: 1.1427x; 1.1427x over previous
import functools

import jax
import jax.numpy as jnp
from jax import lax
from jax.experimental import pallas as pl
from jax.experimental.pallas import tpu as pltpu
from jax.experimental.pallas import tpu_sc as plsc

H = 128
L = 16
NC = 2
NS = 16
NW = NC * NS
C = 64
S = 8

_PERM_DN = jax.lax.GatherDimensionNumbers(
    offset_dims=(), collapsed_slice_dims=(0,), start_index_map=(0,))


def _perm(v, idx):
    return jax.lax.gather(
        v, idx[:, None], _PERM_DN, (1,),
        mode=jax.lax.GatherScatterMode.PROMISE_IN_BOUNDS,
        unique_indices=True, indices_are_sorted=False)


def _inter(v):
    n = v.shape[0]
    return v.reshape(n, H // 32, 2, L).transpose(0, 1, 3, 2).reshape(n, H)


def _matmul_body(x_ref, w1_ref, b1_ref, xa_ref, xb_ref):
    xv = x_ref[...]
    w = w1_ref[...]
    dn = (((1,), (0,)), ((), ()))
    hi = jax.lax.Precision.HIGHEST
    xa_ref[...] = (
        jax.lax.dot_general(xv, w[:H, :], dn, precision=hi) + b1_ref[...]
    )
    xb_ref[...] = jax.lax.dot_general(xv, w[H:, :], dn, precision=hi)


def _ln_body(x_ref, acc_ref, g_ref, b_ref, o_ref):
    a = x_ref[...] + acc_ref[0] + acc_ref[1]
    mean = jnp.mean(a, axis=-1, keepdims=True)
    d = a - mean
    var = jnp.mean(d * d, axis=-1, keepdims=True)
    o_ref[...] = d * jax.lax.rsqrt(var + 1e-5) * g_ref[...] + b_ref[...]


def _make_sc_kernel(n_nodes, acc_rows, k0, k1):
    stripe = acc_rows // NS
    assert k0 % (2 * S) == 0 and k1 % (2 * S) == 0

    mesh = plsc.VectorSubcoreMesh(core_axis_name="c", subcore_axis_name="s")

    @functools.partial(
        pl.kernel,
        out_type=jax.ShapeDtypeStruct((NC, acc_rows, H), jnp.float32),
        mesh=mesh,
        compiler_params=pltpu.CompilerParams(needs_layout_passes=False),
        scratch_types=[
            pltpu.VMEM((2, C, H), jnp.int32),
            pltpu.VMEM((2, C, H), jnp.float32),
            pltpu.VMEM((2, S, C), jnp.int32),
            pltpu.VMEM((2, S, C), jnp.int32),
            pltpu.VMEM((H // 2, ), jnp.int32),
            pltpu.VMEM((L, ), jnp.float32),
            pltpu.VMEM((8, H), jnp.float32),
            pltpu.VMEM_SHARED((acc_rows, H), jnp.float32),
            pltpu.SemaphoreType.DMA,
            pltpu.SemaphoreType.DMA,
            pltpu.SemaphoreType.DMA,
            pltpu.SemaphoreType.DMA,
            pltpu.SemaphoreType.DMA,
            pltpu.SemaphoreType.DMA,
            pltpu.SemaphoreType.DMA,
            pltpu.SemaphoreType.DMA,
        ],
    )
    def sc_edge(xcat_hbm, xb_hbm, src_hbm, dst_hbm, w2_hbm, pv_hbm, acc_out,
                gbuf, bbuf, sslab, dslab, w2v, pv, zbuf, acc,
                sem_pre, sem_slab, sg0, sg1, sb0, sb1, sem_sc, sem_zero):
        cid = lax.axis_index("c")
        sid = lax.axis_index("s")
        k_mine = jnp.where(cid == 0, k0, k1)
        row0 = jnp.where(cid == 0, sid * k0, NS * k0 + sid * k1)
        sem_g = (sg0, sg1)
        sem_b = (sb0, sb1)

        def _issue_slab(m):
            pltpu.async_copy(src_hbm.at[pl.ds(row0 + m * S, S)],
                             sslab.at[lax.rem(m, 2)], sem_slab)
            pltpu.async_copy(dst_hbm.at[pl.ds(row0 + m * S, S)],
                             dslab.at[lax.rem(m, 2)], sem_slab)

        def _wait_slab(m):
            pltpu.make_async_copy(src_hbm.at[pl.ds(row0 + m * S, S)],
                                  sslab.at[lax.rem(m, 2)], sem_slab).wait()
            pltpu.make_async_copy(dst_hbm.at[pl.ds(row0 + m * S, S)],
                                  dslab.at[lax.rem(m, 2)], sem_slab).wait()

        def _sidx(k):
            return sslab.at[lax.rem(k // S, 2), lax.rem(k, S)]

        def _didx(k):
            return dslab.at[lax.rem(k // S, 2), lax.rem(k, S)]

        def _issue_gathers(k, s):
            pltpu.async_copy(xcat_hbm.at[_sidx(k)], gbuf.at[s], sem_g[s])
            pltpu.async_copy(xb_hbm.at[_didx(k)], bbuf.at[s], sem_b[s])

        def _wait_gathers(k, s):
            pltpu.make_async_copy(xcat_hbm.at[_sidx(k)], gbuf.at[s],
                                  sem_g[s]).wait()
            pltpu.make_async_copy(xb_hbm.at[_didx(k)], bbuf.at[s],
                                  sem_b[s]).wait()

        def _wait_scatter(k, par):
            pltpu.make_async_copy(bbuf.at[par], acc.at[_didx(k)],
                                  sem_sc).wait()

        _issue_slab(0)
        pltpu.async_copy(w2_hbm, w2v, sem_pre)
        pltpu.async_copy(pv_hbm, pv, sem_pre)

        zero16 = jnp.zeros((L,), jnp.float32)
        for i in range(8):
            for j8 in range(H // L):
                zbuf[i, pl.ds(j8 * L, L)] = zero16

        _wait_slab(0)
        _issue_gathers(0, 0)

        def _zero_issue(i, _):
            pltpu.async_copy(zbuf, acc.at[pl.ds(sid * stripe + i * 8, 8)],
                             sem_zero)
            return 0

        def _zero_drain(i, _):
            pltpu.make_async_copy(
                zbuf, acc.at[pl.ds(sid * stripe + i * 8, 8)],
                sem_zero).wait()
            return 0

        lax.fori_loop(0, stripe // 8, _zero_issue, 0)
        lax.fori_loop(0, stripe // 8, _zero_drain, 0)

        pltpu.make_async_copy(w2_hbm, w2v, sem_pre).wait()
        pltpu.make_async_copy(pv_hbm, pv, sem_pre).wait()
        plsc.subcore_barrier()

        b2s = pv[pl.ds(0, L)][0]
        lanes = lax.iota(jnp.int32, L)
        w2blk = [plsc.bitcast(w2v[pl.ds(g * L, L)], jnp.bfloat16)
                 for g in range(H // 32)]

        def _pair_body(kk, _):
            for s in range(2):
                k = kk * 2 + s
                @pl.when(k >= 1)
                def _():
                    _wait_scatter(k - 1, 1 - s)

                @pl.when(jnp.logical_and(lax.rem(k, S) == 0,
                                         k + S < k_mine))
                def _():
                    _issue_slab(k // S + 1)

                @pl.when(k + 1 < k_mine)
                def _():
                    @pl.when(lax.rem(k + 1, S) == 0)
                    def _():
                        _wait_slab((k + 1) // S)

                    _issue_gathers(k + 1, 1 - s)

                _wait_gathers(k, s)

                g2 = gbuf.at[s]
                bb = bbuf.at[s]

                def _zgroup(gr, _):
                    e0 = gr * L
                    zv = jnp.zeros((L,), jnp.float32)
                    for i in range(L):
                        e = e0 + i
                        sacc = None
                        for g in range(H // 32):
                            av = plsc.bitcast(g2[e, pl.ds(g * L, L)],
                                              jnp.bfloat16)
                            blo = bb[e, pl.ds(g * 32, L)]
                            bhi = bb[e, pl.ds(g * 32 + L, L)]
                            bv = plsc.pack(
                                blo, bhi, format=plsc.PackFormat.INTERLEAVED)
                            t = jnp.maximum(av + bv, 0) * w2blk[g]
                            sacc = t if sacc is None else sacc + t
                        sa, sb = plsc.unpack(
                            sacc, format=plsc.PackFormat.INTERLEAVED)
                        sv = sa + sb
                        for sh in (8, 4, 2, 1):
                            sv = sv + _perm(sv, lanes ^ sh)
                        zv = zv + jnp.where(lanes == i, sv, 0.0)
                    wgt = 1.0 / (1.0 + jnp.exp(-(zv + b2s)))
                    for i in range(L):
                        e = e0 + i
                        we = wgt[i]
                        for g in range(H // 32):
                            v32 = plsc.bitcast(
                                g2[e, pl.ds(H // 2 + g * L, L)], jnp.bfloat16)
                            va, vb = plsc.unpack(
                                v32, format=plsc.PackFormat.INTERLEAVED)
                            bb[e, pl.ds(g * 32, L)] = va * we
                            bb[e, pl.ds(g * 32 + L, L)] = vb * we
                    return 0

                lax.fori_loop(0, C // L, _zgroup, 0)

                pltpu.async_copy(bb, acc.at[_didx(k)], sem_sc, add=True)
            return 0

        lax.fori_loop(0, k_mine // 2, _pair_body, 0)

        _wait_scatter(k_mine - 1, 1)
        plsc.subcore_barrier()
        pltpu.sync_copy(
            acc.at[pl.ds(sid * stripe, stripe)],
            acc_out.at[cid, pl.ds(sid * stripe, stripe)],
        )

    return sc_edge


def kernel(x, edge_index, W1, b1, W2, b2, gamma, beta):
    n_nodes = x.shape[0]
    n_edges = edge_index.shape[1]

    src = edge_index[0].astype(jnp.int32)
    dst = edge_index[1].astype(jnp.int32)

    k_chunks = -(-n_edges // (NW * C))
    k_chunks += -k_chunks % (2 * S)
    k0 = ((2 * k_chunks * 65) // 100 // (2 * S)) * (2 * S)
    k1 = 2 * k_chunks - k0
    e_pad = NW * k_chunks * C - n_edges
    srcp = jnp.concatenate([src, jnp.zeros((e_pad,), jnp.int32)])
    dstp = jnp.concatenate([dst, jnp.full((e_pad,), n_nodes, jnp.int32)])
    srcp = srcp.reshape(NW * k_chunks, C)
    dstp = dstp.reshape(NW * k_chunks, C)
    del n_edges

    acc_rows = n_nodes + (-n_nodes % (NS * 8)) + (NS * 8 if n_nodes % (NS * 8) == 0 and e_pad else 0)

    br = 400
    grid = n_nodes // br
    xa, xb = pl.pallas_call(
        _matmul_body,
        grid=(grid,),
        in_specs=[
            pl.BlockSpec((br, H), lambda i: (i, 0)),
            pl.BlockSpec((2 * H, H), lambda i: (0, 0)),
            pl.BlockSpec((1, H), lambda i: (0, 0)),
        ],
        out_specs=[
            pl.BlockSpec((br, H), lambda i: (i, 0)),
            pl.BlockSpec((br, H), lambda i: (i, 0)),
        ],
        out_shape=[
            jax.ShapeDtypeStruct((n_nodes, H), jnp.float32),
            jax.ShapeDtypeStruct((n_nodes, H), jnp.float32),
        ],
    )(x, W1, b1.reshape(1, H))

    def _as_i32(v):
        n = v.shape[0]
        return jax.lax.bitcast_convert_type(
            v.astype(jnp.bfloat16).reshape(n, v.shape[1] // 2, 2), jnp.int32)

    xcat = _as_i32(jnp.concatenate([_inter(xa), _inter(x)], axis=1))
    w2p = _as_i32(_inter(W2.reshape(1, H)))[0]
    pvv = jnp.concatenate([b2.reshape(1), jnp.zeros((L - 1,), jnp.float32)])

    sc_edge = _make_sc_kernel(n_nodes, acc_rows, k0, k1)
    acc = sc_edge(xcat, xb, srcp, dstp, w2p, pvv)

    out = pl.pallas_call(
        _ln_body,
        grid=(grid,),
        in_specs=[
            pl.BlockSpec((br, H), lambda i: (i, 0)),
            pl.BlockSpec((NC, br, H), lambda i: (0, i, 0)),
            pl.BlockSpec((1, H), lambda i: (0, 0)),
            pl.BlockSpec((1, H), lambda i: (0, 0)),
        ],
        out_specs=pl.BlockSpec((br, H), lambda i: (i, 0)),
        out_shape=jax.ShapeDtypeStruct((n_nodes, H), jnp.float32),
    )(x, acc, gamma.reshape(1, H), beta.reshape(1, H))

    return out
